# baseline (speedup 1.0000x reference)
import functools
import math

import jax
import jax.numpy as jnp
import numpy as np
from jax import lax
from jax.experimental import pallas as pl
from jax.experimental.pallas import tpu as pltpu
from jax.experimental.pallas import tpu_sc as plsc

VOCAB = 100000
D_MODEL = 768
MAX_SEQ_LEN = 8192
BATCH = 4
SEQ_LEN = 4096
SCALE = math.sqrt(D_MODEL)

NUM_CORES = 2
NUM_SUBCORES = 16
NW = NUM_CORES * NUM_SUBCORES
POS_PER_W = SEQ_LEN // NW
CHUNK = 64
N_PCHUNK = POS_PER_W // CHUNK
LANES = 16
VPR = D_MODEL // LANES


def _pos_encoding() -> np.ndarray:
    pos = np.arange(SEQ_LEN)[:, None].astype(np.float32)
    div = np.exp(
        np.arange(0, D_MODEL, 2).astype(np.float32)
        * (-math.log(10000.0) / D_MODEL)
    )
    pe = np.zeros((SEQ_LEN, D_MODEL), dtype=np.float32)
    pe[:, 0::2] = np.sin(pos * div)
    pe[:, 1::2] = np.cos(pos * div)
    return pe


_PE = _pos_encoding()


@functools.partial(
    pl.kernel,
    out_type=jax.ShapeDtypeStruct((BATCH * SEQ_LEN, D_MODEL), jnp.float32),
    mesh=plsc.VectorSubcoreMesh(core_axis_name="c", subcore_axis_name="s"),
    scratch_types=[
        pltpu.VMEM((CHUNK,), jnp.int32),
        pltpu.VMEM((CHUNK, D_MODEL), jnp.float32),
        pltpu.VMEM((CHUNK, D_MODEL), jnp.float32),
        pltpu.SemaphoreType.DMA,
    ],
)
def _embed(ids_hbm, table_hbm, pe_hbm, out_hbm, idx_v, pe_v, rows_v, sem):
    wid = lax.axis_index("s") * NUM_CORES + lax.axis_index("c")
    pos_base = wid * POS_PER_W

    for pc in range(N_PCHUNK):
        pos_off = pos_base + pc * CHUNK
        pltpu.sync_copy(pe_hbm.at[pl.ds(pos_off, CHUNK)], pe_v)
        for b in range(BATCH):
            row_off = b * SEQ_LEN + pos_off
            pltpu.sync_copy(ids_hbm.at[pl.ds(row_off, CHUNK)], idx_v)
            pltpu.async_copy(table_hbm.at[idx_v], rows_v, sem).wait()

            def body(r, carry):
                for j in range(VPR):
                    sl = pl.ds(j * LANES, LANES)
                    rows_v[r, sl] = rows_v[r, sl] * SCALE + pe_v[r, sl]
                return carry

            lax.fori_loop(0, CHUNK, body, 0)
            pltpu.sync_copy(rows_v, out_hbm.at[pl.ds(row_off, CHUNK)])


def kernel(token_ids, table):
    ids_flat = token_ids.reshape(-1).astype(jnp.int32)
    out = _embed(ids_flat, table, jnp.asarray(_PE))
    return out.reshape(BATCH, SEQ_LEN, D_MODEL)


# speedup vs baseline: 1.3746x; 1.3746x over previous
import functools
import math

import jax
import jax.numpy as jnp
import numpy as np
from jax import lax
from jax.experimental import pallas as pl
from jax.experimental.pallas import tpu as pltpu
from jax.experimental.pallas import tpu_sc as plsc

VOCAB = 100000
D_MODEL = 768
BATCH = 4
SEQ_LEN = 4096
SCALE = math.sqrt(D_MODEL)

NUM_CORES = 2
NUM_SUBCORES = 16
NW = NUM_CORES * NUM_SUBCORES
POS_PER_W = SEQ_LEN // NW
CHUNK = 32
N_PCHUNK = POS_PER_W // CHUNK
NCHUNK = N_PCHUNK * BATCH
LANES = 16
VPR = D_MODEL // LANES
NBUF = 3


def _pos_encoding() -> np.ndarray:
    pos = np.arange(SEQ_LEN)[:, None].astype(np.float32)
    div = np.exp(
        np.arange(0, D_MODEL, 2).astype(np.float32)
        * (-math.log(10000.0) / D_MODEL)
    )
    pe = np.zeros((SEQ_LEN, D_MODEL), dtype=np.float32)
    pe[:, 0::2] = np.sin(pos * div)
    pe[:, 1::2] = np.cos(pos * div)
    return pe


_PE = _pos_encoding()


@functools.partial(
    pl.kernel,
    out_type=jax.ShapeDtypeStruct((BATCH * SEQ_LEN, D_MODEL), jnp.float32),
    mesh=plsc.VectorSubcoreMesh(core_axis_name="c", subcore_axis_name="s"),
    scratch_types=[
        pltpu.VMEM((BATCH, POS_PER_W), jnp.int32),
        pltpu.VMEM((CHUNK, D_MODEL), jnp.float32),
        pltpu.VMEM((CHUNK, D_MODEL), jnp.float32),
        pltpu.VMEM((CHUNK, D_MODEL), jnp.float32),
        pltpu.VMEM((CHUNK, D_MODEL), jnp.float32),
        pltpu.VMEM((CHUNK, D_MODEL), jnp.float32),
        pltpu.SemaphoreType.DMA,
        pltpu.SemaphoreType.DMA,
        pltpu.SemaphoreType.DMA,
        pltpu.SemaphoreType.DMA,
        pltpu.SemaphoreType.DMA,
        pltpu.SemaphoreType.DMA,
        pltpu.SemaphoreType.DMA,
        pltpu.SemaphoreType.DMA,
    ],
)
def _embed(ids_hbm, table_hbm, pe_hbm, out_hbm,
           idx_all, pe0, pe1, r0, r1, r2,
           g0, g1, g2, w0, w1, w2, p0, p1):
    wid = lax.axis_index("s") * NUM_CORES + lax.axis_index("c")
    pos_base = wid * POS_PER_W

    rows = (r0, r1, r2)
    gsem = (g0, g1, g2)
    wsem = (w0, w1, w2)
    pebuf = (pe0, pe1)
    psem = (p0, p1)

    pltpu.sync_copy(ids_hbm.at[:, pl.ds(pos_base, POS_PER_W)], idx_all)
    pltpu.sync_copy(pe_hbm.at[pl.ds(pos_base, CHUNK)], pebuf[0])

    copies_g = [None] * NCHUNK
    copies_w = [None] * NCHUNK
    copies_p = [None] * N_PCHUNK

    if N_PCHUNK > 1:
        copies_p[1] = pltpu.async_copy(
            pe_hbm.at[pl.ds(pos_base + CHUNK, CHUNK)], pebuf[1], psem[1])

    def compute(rbuf, pbuf):
        def body(r, carry):
            for j in range(VPR):
                sl = pl.ds(j * LANES, LANES)
                rbuf[r, sl] = rbuf[r, sl] * SCALE + pbuf[r, sl]
            return carry
        lax.fori_loop(0, CHUNK, body, 0)

    for c in range(NCHUNK + 1):
        if c < NCHUNK:
            pc, b = divmod(c, BATCH)
            slot = c % NBUF
            if c >= NBUF:
                copies_w[c - NBUF].wait()
            copies_g[c] = pltpu.async_copy(
                table_hbm.at[idx_all.at[b, pl.ds(pc * CHUNK, CHUNK)]],
                rows[slot], gsem[slot])
        if c >= 1:
            cc = c - 1
            pcc, bcc = divmod(cc, BATCH)
            cslot = cc % NBUF
            if bcc == 0 and pcc >= 1:
                copies_p[pcc].wait()
            copies_g[cc].wait()
            compute(rows[cslot], pebuf[pcc % 2])
            row_off = bcc * SEQ_LEN + pos_base + pcc * CHUNK
            copies_w[cc] = pltpu.async_copy(
                rows[cslot], out_hbm.at[pl.ds(row_off, CHUNK)], wsem[cslot])
            if bcc == BATCH - 1 and pcc + 2 < N_PCHUNK:
                npc = pcc + 2
                copies_p[npc] = pltpu.async_copy(
                    pe_hbm.at[pl.ds(pos_base + npc * CHUNK, CHUNK)],
                    pebuf[npc % 2], psem[npc % 2])

    for c in range(NCHUNK - NBUF, NCHUNK):
        copies_w[c].wait()


def kernel(token_ids, table):
    ids = token_ids.astype(jnp.int32)
    out = _embed(ids, table, jnp.asarray(_PE))
    return out.reshape(BATCH, SEQ_LEN, D_MODEL)
